# baseline (speedup 1.0000x reference)
import functools

import jax
import jax.numpy as jnp
from jax.experimental import pallas as pl
from jax.experimental.pallas import tpu as pltpu

_BN_EPS = 1e-5
_NORM_EPS = 1e-12
_SLOPE = 0.1

_NT = (((1,), (1,)), ((), ()))


def _pool_mm_kernel(x_ref, w_ref, pda_ref, pooled_ref, bpart_ref, *, inv_hw):
    h = pl.program_id(1)

    @pl.when(h == 0)
    def _():
        pooled_ref[...] = jnp.zeros_like(pooled_ref)

    xt = x_ref[...]
    pda_ref[...] = xt
    pooled_ref[...] += jnp.sum(xt, axis=0)

    @pl.when(h == pl.num_programs(1) - 1)
    def _():
        p = pooled_ref[...] * inv_hw
        pooled_ref[...] = p
        bpart_ref[0] = jax.lax.dot_general(
            p.astype(jnp.bfloat16), w_ref[...].astype(jnp.bfloat16), _NT,
            preferred_element_type=jnp.float32)


def _head_kernel(bp_ref, gamma_ref, wcls_ref, ctr_ref,
                 bn_ref, cls_ref, logit_ref, dist_ref):
    f32 = jnp.float32
    b = jnp.sum(bp_ref[...], axis=0)
    b = jnp.where(b >= 0, b, _SLOPE * b)
    mu = jnp.mean(b, axis=0, keepdims=True)
    var = jnp.mean((b - mu) ** 2, axis=0, keepdims=True)
    bn = (b - mu) * jax.lax.rsqrt(var + _BN_EPS) * gamma_ref[...]
    bn_ref[...] = bn

    x2 = jnp.sum(bn * bn, axis=1, keepdims=True)
    xinv = jax.lax.rsqrt(jnp.maximum(x2, _NORM_EPS * _NORM_EPS))

    ones_row = jnp.ones((1, bn.shape[1]), f32)
    wcls = wcls_ref[...]
    winv = jax.lax.rsqrt(jnp.maximum(
        jax.lax.dot_general(ones_row, wcls * wcls, _NT,
                            preferred_element_type=f32),
        _NORM_EPS * _NORM_EPS))

    cls = jax.lax.dot_general(bn.astype(jnp.bfloat16),
                              wcls.astype(jnp.bfloat16), _NT,
                              preferred_element_type=f32)
    cls_ref[...] = cls
    logit_ref[...] = cls * xinv * winv

    ctr = ctr_ref[...]
    c2 = jax.lax.dot_general(ones_row, ctr * ctr, _NT,
                             preferred_element_type=f32)
    dist_ref[...] = x2 + c2 - 2.0 * jax.lax.dot_general(
        bn, ctr, _NT, preferred_element_type=f32)


def kernel(features, w_fc, gamma, w_cls, centers):
    f32 = jnp.float32
    N, C, H, W = features.shape
    R = w_fc.shape[0]
    K = w_cls.shape[0]
    HW = H * W

    xt = features.reshape(N, C, HW).transpose(2, 0, 1)
    tc = next((t for t in (1024, 512, 256, 128) if C % t == 0), C)
    P = C // tc
    thw = next((t for t in (28, 16, 14, 8, 7, 4, 2) if HW % t == 0), 1)

    pda3, pooled, b_parts = pl.pallas_call(
        functools.partial(_pool_mm_kernel, inv_hw=1.0 / HW),
        out_shape=(jax.ShapeDtypeStruct((HW, N, C), f32),
                   jax.ShapeDtypeStruct((N, C), f32),
                   jax.ShapeDtypeStruct((P, N, R), f32)),
        grid=(P, HW // thw),
        in_specs=[pl.BlockSpec((thw, N, tc), lambda c, h: (h, 0, c)),
                  pl.BlockSpec((R, tc), lambda c, h: (0, c))],
        out_specs=(pl.BlockSpec((thw, N, tc), lambda c, h: (h, 0, c)),
                   pl.BlockSpec((N, tc), lambda c, h: (0, c)),
                   pl.BlockSpec((1, N, R), lambda c, h: (c, 0, 0))),
        compiler_params=pltpu.CompilerParams(
            dimension_semantics=("parallel", "arbitrary"),
            vmem_limit_bytes=64 * 1024 * 1024),
    )(xt, w_fc)

    bn_feat, cls_o, logits_o, dist_o = pl.pallas_call(
        _head_kernel,
        out_shape=(jax.ShapeDtypeStruct((N, R), f32),
                   jax.ShapeDtypeStruct((N, K), f32),
                   jax.ShapeDtypeStruct((N, K), f32),
                   jax.ShapeDtypeStruct((N, K), f32)),
        compiler_params=pltpu.CompilerParams(
            vmem_limit_bytes=64 * 1024 * 1024),
    )(b_parts, gamma, w_cls, centers)

    return {
        "pda_features": pda3.transpose(1, 2, 0).reshape(N, C, H, W),
        "cls_outputs": cls_o,
        "pred_class_logits": logits_o,
        "pooled_features": pooled,
        "bn_features": bn_feat,
        "center_distmat": dist_o,
    }


# speedup vs baseline: 1.0589x; 1.0589x over previous
import functools

import jax
import jax.numpy as jnp
from jax.experimental import pallas as pl
from jax.experimental.pallas import tpu as pltpu

_BN_EPS = 1e-5
_NORM_EPS = 1e-12
_SLOPE = 0.1

_NT = (((1,), (1,)), ((), ()))


def _pool_mm_kernel(x_ref, w_ref, pda_ref, pooled_ref, bpart_ref, *, inv_hw):
    h = pl.program_id(1)

    @pl.when(h == 0)
    def _():
        pooled_ref[...] = jnp.zeros_like(pooled_ref)

    xt = x_ref[...]
    pda_ref[...] = xt
    pooled_ref[...] += jnp.sum(xt, axis=0)

    @pl.when(h == pl.num_programs(1) - 1)
    def _():
        p = pooled_ref[...] * inv_hw
        pooled_ref[...] = p
        bpart_ref[0] = jax.lax.dot_general(
            p.astype(jnp.bfloat16), w_ref[...].astype(jnp.bfloat16), _NT,
            preferred_element_type=jnp.float32)


def _head_kernel(bp_ref, gamma_ref, wcls_ref, ctr_ref,
                 bn_ref, cls_ref, logit_ref, dist_ref):
    f32 = jnp.float32
    b = jnp.sum(bp_ref[...], axis=0)
    b = jnp.where(b >= 0, b, _SLOPE * b)
    mu = jnp.mean(b, axis=0, keepdims=True)
    var = jnp.mean((b - mu) ** 2, axis=0, keepdims=True)
    bn = (b - mu) * jax.lax.rsqrt(var + _BN_EPS) * gamma_ref[...]
    bn_ref[...] = bn

    x2 = jnp.sum(bn * bn, axis=1, keepdims=True)
    xinv = jax.lax.rsqrt(jnp.maximum(x2, _NORM_EPS * _NORM_EPS))

    ones_row = jnp.ones((1, bn.shape[1]), f32)
    wcls = wcls_ref[...]
    winv = jax.lax.rsqrt(jnp.maximum(
        jax.lax.dot_general(ones_row, wcls * wcls, _NT,
                            preferred_element_type=f32),
        _NORM_EPS * _NORM_EPS))

    cls = jax.lax.dot_general(bn.astype(jnp.bfloat16),
                              wcls.astype(jnp.bfloat16), _NT,
                              preferred_element_type=f32)
    cls_ref[...] = cls
    logit_ref[...] = cls * xinv * winv

    ctr = ctr_ref[...]
    c2 = jax.lax.dot_general(ones_row, ctr * ctr, _NT,
                             preferred_element_type=f32)
    dist_ref[...] = x2 + c2 - 2.0 * jax.lax.dot_general(
        bn, ctr, _NT, preferred_element_type=f32)


def kernel(features, w_fc, gamma, w_cls, centers):
    f32 = jnp.float32
    N, C, H, W = features.shape
    R = w_fc.shape[0]
    K = w_cls.shape[0]
    HW = H * W

    xt = features.reshape(N, C, HW).transpose(2, 0, 1)
    tc = next((t for t in (1024, 512, 256, 128) if C % t == 0), C)
    P = C // tc
    thw = next((t for t in (98, 49, 28, 16, 14, 8, 7, 4, 2) if HW % t == 0), 1)

    pda3, pooled, b_parts = pl.pallas_call(
        functools.partial(_pool_mm_kernel, inv_hw=1.0 / HW),
        out_shape=(jax.ShapeDtypeStruct((HW, N, C), f32),
                   jax.ShapeDtypeStruct((N, C), f32),
                   jax.ShapeDtypeStruct((P, N, R), f32)),
        grid=(P, HW // thw),
        in_specs=[pl.BlockSpec((thw, N, tc), lambda c, h: (h, 0, c)),
                  pl.BlockSpec((R, tc), lambda c, h: (0, c))],
        out_specs=(pl.BlockSpec((thw, N, tc), lambda c, h: (h, 0, c)),
                   pl.BlockSpec((N, tc), lambda c, h: (0, c)),
                   pl.BlockSpec((1, N, R), lambda c, h: (c, 0, 0))),
        compiler_params=pltpu.CompilerParams(
            dimension_semantics=("parallel", "arbitrary"),
            vmem_limit_bytes=64 * 1024 * 1024),
    )(xt, w_fc)

    bn_feat, cls_o, logits_o, dist_o = pl.pallas_call(
        _head_kernel,
        out_shape=(jax.ShapeDtypeStruct((N, R), f32),
                   jax.ShapeDtypeStruct((N, K), f32),
                   jax.ShapeDtypeStruct((N, K), f32),
                   jax.ShapeDtypeStruct((N, K), f32)),
        compiler_params=pltpu.CompilerParams(
            vmem_limit_bytes=64 * 1024 * 1024),
    )(b_parts, gamma, w_cls, centers)

    return {
        "pda_features": pda3.transpose(1, 2, 0).reshape(N, C, H, W),
        "cls_outputs": cls_o,
        "pred_class_logits": logits_o,
        "pooled_features": pooled,
        "bn_features": bn_feat,
        "center_distmat": dist_o,
    }
